# baseline (speedup 1.0000x reference)
import functools

import jax
import jax.numpy as jnp
from jax import lax
from jax.experimental import pallas as pl
from jax.experimental.pallas import tpu as pltpu
from jax.experimental.pallas import tpu_sc as plsc

N = 10000
E = 320000
D = 128
H = 4
OC = D // H
NEG = 0.02

NC = 2
NS = 16
NW = NC * NS
EW = E // NW
K = 80
NCHUNK = EW // K
ROW = D + 16
RPT = N // NS
RZB = 125
SUP = 20
NSUP = 7


def _scores_body(x_ref, w_ref, s_ref):
    s_ref[...] = jnp.dot(x_ref[...], w_ref[...],
                         preferred_element_type=jnp.float32)


def _scores(x, wfull):
    return pl.pallas_call(
        _scores_body,
        out_shape=jax.ShapeDtypeStruct((N, 16), jnp.float32),
    )(x, wfull)


def _edge_body(x_hbm, s_hbm, ei_hbm, ej_hbm, tmax_hbm, out_hbm,
               ivbig, jvbig, sia, sja, xja, sib, sjb, xjb,
               wx, wtmp, tmaxv, acc,
               sa1, sa2, sa3, sb1, sb2, sb3, sems):
    cid = lax.axis_index("c")
    sid = lax.axis_index("s")
    wid = cid * NS + sid

    zero16 = jnp.zeros((16,), jnp.float32)

    @pl.loop(0, K)
    def _(r):
        for c in range(ROW // 16):
            wx[r, pl.ds(c * 16, 16)] = zero16

    rz = sid * RPT

    for g in range(RPT // K):
        pltpu.async_copy(wx, acc.at[pl.ds(rz + g * K, K)], sems)

    pltpu.async_copy(wx.at[pl.ds(0, RPT % K)],
                     acc.at[pl.ds(rz + (RPT // K) * K, RPT % K)], sems)

    for g in range(RPT // K):
        pltpu.make_async_copy(wx, acc.at[pl.ds(rz + g * K, K)], sems).wait()

    pltpu.make_async_copy(wx.at[pl.ds(0, RPT % K)],
                          acc.at[pl.ds(rz + (RPT // K) * K, RPT % K)],
                          sems).wait()

    pltpu.sync_copy(tmax_hbm, tmaxv)
    plsc.subcore_barrier()

    base = wid * (EW // K)
    iota16 = lax.iota(jnp.int32, 16)

    bufs = ((sia, sja, xja, sa1, sa2, sa3),
            (sib, sjb, xjb, sb1, sb2, sb3))

    def fire(q, b):
        six, sjx, xjx, s1, s2, s3 = bufs[b]
        pltpu.async_copy(s_hbm.at[ivbig.at[q]], six, s1)
        pltpu.async_copy(s_hbm.at[jvbig.at[q]], sjx, s2)
        pltpu.async_copy(x_hbm.at[jvbig.at[q]], xjx, s3)

    def wait_gathers(b):
        six, sjx, xjx, s1, s2, s3 = bufs[b]
        pltpu.make_async_copy(s_hbm.at[ivbig.at[0]], six, s1).wait()
        pltpu.make_async_copy(s_hbm.at[jvbig.at[0]], sjx, s2).wait()
        pltpu.make_async_copy(x_hbm.at[jvbig.at[0]], xjx, s3).wait()

    def scatter_drain():
        pltpu.make_async_copy(wx, acc.at[ivbig.at[0]], sems).wait()

    def compute_and_scatter(b, q, cnt):
        six, sjx, xjx, s1, s2, s3 = bufs[b]

        @pl.when(q > 0)
        def _():
            scatter_drain()
        @pl.loop(0, K // 16)
        def _(g):
            lane = iota16 + g * 16
            for h in range(H):
                s_v = plsc.load_gather(six, [lane, jnp.full((16,), 8 + h, jnp.int32)])
                t_v = plsc.load_gather(sjx, [lane, jnp.full((16,), 12 + h, jnp.int32)])
                tm = tmaxv[h]
                u = s_v + t_v
                a = jnp.maximum(u, u * NEG)
                sm = s_v + tm
                m = jnp.maximum(sm, sm * NEG)
                w = jnp.exp(a - m)
                plsc.store_scatter(wx, [lane, jnp.full((16,), D + h, jnp.int32)], w)
                wtmp[pl.ds(16 + h * 16, 16)] = w

            @plsc.parallel_loop(0, 16, unroll=8)
            def _(l):
                e = g * 16 + l
                for h in range(H):
                    wb = plsc.load_gather(
                        wtmp, [jnp.full((16,), 16 + h * 16, jnp.int32) + l])
                    for p in range(2):
                        col = h * OC + p * 16
                        wx[e, pl.ds(col, 16)] = xjx[e, pl.ds(col, 16)] * wb

        pltpu.async_copy(wx, acc.at[ivbig.at[q]], sems, add=True)

        @pl.when(q == cnt - 1)
        def _():
            scatter_drain()

    @pl.loop(0, NSUP)
    def _(sup):
        cnt = jnp.where(sup < NSUP - 1, SUP, NCHUNK - (NSUP - 1) * SUP)
        roff = base + sup * SUP
        pltpu.sync_copy(ei_hbm.at[pl.ds(roff, SUP)], ivbig)
        pltpu.sync_copy(ej_hbm.at[pl.ds(roff, SUP)], jvbig)
        fire(0, 0)
        fire(1, 1)

        @pl.loop(0, (cnt + 1) // 2)
        def _(i):
            c0 = 2 * i
            wait_gathers(0)
            compute_and_scatter(0, c0, cnt)

            @pl.when(c0 + 2 < cnt)
            def _():
                fire(c0 + 2, 0)

            @pl.when(c0 + 1 < cnt)
            def _():
                wait_gathers(1)
                compute_and_scatter(1, c0 + 1, cnt)

                @pl.when(c0 + 3 < cnt)
                def _():
                    fire(c0 + 3, 1)

    plsc.subcore_barrier()

    r0 = sid * RPT

    for g in range(5):
        pltpu.async_copy(acc.at[pl.ds(r0 + g * RZB, RZB)],
                         out_hbm.at[cid, pl.ds(r0 + g * RZB, RZB)], sems)

    for g in range(5):
        pltpu.make_async_copy(acc.at[pl.ds(r0 + g * RZB, RZB)],
                              out_hbm.at[cid, pl.ds(r0 + g * RZB, RZB)],
                              sems).wait()


def _edge_kernel(x, s_tab, ei, ej, tmax16):
    mesh = plsc.VectorSubcoreMesh(core_axis_name="c", subcore_axis_name="s")
    run = functools.partial(
        pl.kernel,
        out_type=jax.ShapeDtypeStruct((NC, N, ROW), jnp.float32),
        mesh=mesh,
        compiler_params=pltpu.CompilerParams(
            use_tc_tiling_on_sc=False, needs_layout_passes=False),
        scratch_types=[
            pltpu.VMEM((SUP, K), jnp.int32),
            pltpu.VMEM((SUP, K), jnp.int32),
            pltpu.VMEM((K, 16), jnp.float32),
            pltpu.VMEM((K, 16), jnp.float32),
            pltpu.VMEM((K, D), jnp.float32),
            pltpu.VMEM((K, 16), jnp.float32),
            pltpu.VMEM((K, 16), jnp.float32),
            pltpu.VMEM((K, D), jnp.float32),
            pltpu.VMEM((K, ROW), jnp.float32),
            pltpu.VMEM((96,), jnp.float32),
            pltpu.VMEM((4, 16), jnp.float32),
            pltpu.VMEM_SHARED((N, ROW), jnp.float32),
            pltpu.SemaphoreType.DMA,
            pltpu.SemaphoreType.DMA,
            pltpu.SemaphoreType.DMA,
            pltpu.SemaphoreType.DMA,
            pltpu.SemaphoreType.DMA,
            pltpu.SemaphoreType.DMA,
            pltpu.SemaphoreType.DMA,
        ],
    )(_edge_body)
    return run(x, s_tab, ei, ej, tmax16)


def _norm_body(p_ref, o_ref):
    p = p_ref[...]
    q = p[0] + p[1]
    parts = []
    for h in range(H):
        den = q[:, D + h][:, None] + 1e-16
        parts.append(q[:, h * OC:(h + 1) * OC] / den)
    o_ref[...] = jnp.concatenate(parts, axis=1)


def _norm(p):
    R = 1000
    return pl.pallas_call(
        _norm_body,
        grid=(N // R,),
        in_specs=[pl.BlockSpec((2, R, ROW), lambda b: (0, b, 0))],
        out_specs=pl.BlockSpec((R, D), lambda b: (b, 0)),
        out_shape=jax.ShapeDtypeStruct((N, D), jnp.float32),
    )(p)


def kernel(x_tangent0, edges, W):
    x = x_tangent0
    w1 = W[0, :OC]
    w2 = W[0, OC:]
    eye = jnp.eye(H, dtype=jnp.float32)
    wf_s = jnp.kron(eye, w1[:, None])
    wf_t = jnp.kron(eye, w2[:, None])
    wfull = jnp.concatenate(
        [jnp.zeros((D, 8), jnp.float32), wf_s, wf_t], axis=1)

    s_tab = _scores(x, wfull)
    tmax = jnp.max(s_tab[:, 3 * H:4 * H], axis=0)
    tmax16 = jnp.broadcast_to(tmax[:, None], (H, 16))

    pad = jnp.zeros((SUP - (NCHUNK % SUP or SUP), K), jnp.int32)
    ei = jnp.concatenate([edges[0].astype(jnp.int32).reshape(E // K, K), pad])
    ej = jnp.concatenate([edges[1].astype(jnp.int32).reshape(E // K, K), pad])
    p = _edge_kernel(x, s_tab, ei, ej, tmax16)
    return _norm(p)


# speedup vs baseline: 1.0454x; 1.0454x over previous
import functools

import jax
import jax.numpy as jnp
from jax import lax
from jax.experimental import pallas as pl
from jax.experimental.pallas import tpu as pltpu
from jax.experimental.pallas import tpu_sc as plsc

N = 10000
E = 320000
D = 128
H = 4
OC = D // H
NEG = 0.02

NC = 2
NS = 16
NW = NC * NS
EW = E // NW
K = 80
NCHUNK = EW // K
ROW = D + 16
RPT = N // NS
RZB = 125
SUP = 20
NSUP = 7


def _scores_body(x_ref, w_ref, s_ref):
    s_ref[...] = jnp.dot(x_ref[...], w_ref[...],
                         preferred_element_type=jnp.float32)


def _scores(x, wfull):
    return pl.pallas_call(
        _scores_body,
        out_shape=jax.ShapeDtypeStruct((N, 16), jnp.float32),
    )(x, wfull)


def _edge_body(x_hbm, s_hbm, ei_hbm, ej_hbm, tmax_hbm, out_hbm,
               ivbig, jvbig, sia, sja, xja, sib, sjb, xjb,
               wx, wtmp, tmaxv, acc,
               sa1, sa2, sa3, sb1, sb2, sb3, sems):
    cid = lax.axis_index("c")
    sid = lax.axis_index("s")
    wid = cid * NS + sid

    zero16 = jnp.zeros((16,), jnp.float32)

    @pl.loop(0, K)
    def _(r):
        for c in range(ROW // 16):
            wx[r, pl.ds(c * 16, 16)] = zero16

    rz = sid * RPT

    for g in range(RPT // K):
        pltpu.async_copy(wx, acc.at[pl.ds(rz + g * K, K)], sems)

    pltpu.async_copy(wx.at[pl.ds(0, RPT % K)],
                     acc.at[pl.ds(rz + (RPT // K) * K, RPT % K)], sems)

    for g in range(RPT // K):
        pltpu.make_async_copy(wx, acc.at[pl.ds(rz + g * K, K)], sems).wait()

    pltpu.make_async_copy(wx.at[pl.ds(0, RPT % K)],
                          acc.at[pl.ds(rz + (RPT // K) * K, RPT % K)],
                          sems).wait()

    pltpu.sync_copy(tmax_hbm, tmaxv)
    plsc.subcore_barrier()

    base = wid * (EW // K)
    iota16 = lax.iota(jnp.int32, 16)

    bufs = ((sia, sja, xja, sa1, sa2, sa3),
            (sib, sjb, xjb, sb1, sb2, sb3))

    def fire(q, b):
        six, sjx, xjx, s1, s2, s3 = bufs[b]
        pltpu.async_copy(s_hbm.at[ivbig.at[q]], six, s1)
        pltpu.async_copy(s_hbm.at[jvbig.at[q]], sjx, s2)
        pltpu.async_copy(x_hbm.at[jvbig.at[q]], xjx, s3)

    def wait_gathers(b):
        six, sjx, xjx, s1, s2, s3 = bufs[b]
        pltpu.make_async_copy(s_hbm.at[ivbig.at[0]], six, s1).wait()
        pltpu.make_async_copy(s_hbm.at[jvbig.at[0]], sjx, s2).wait()
        pltpu.make_async_copy(x_hbm.at[jvbig.at[0]], xjx, s3).wait()

    def scatter_drain():
        pltpu.make_async_copy(wx, acc.at[ivbig.at[0]], sems).wait()

    def compute_and_scatter(b, q, cnt):
        six, sjx, xjx, s1, s2, s3 = bufs[b]

        @pl.when(q > 0)
        def _():
            scatter_drain()
        @pl.loop(0, K // 16)
        def _(g):
            lane = iota16 + g * 16
            for h in range(H):
                s_v = plsc.load_gather(six, [lane, jnp.full((16,), 8 + h, jnp.int32)])
                t_v = plsc.load_gather(sjx, [lane, jnp.full((16,), 12 + h, jnp.int32)])
                tm = tmaxv[h]
                u = s_v + t_v
                a = jnp.maximum(u, u * NEG)
                sm = s_v + tm
                m = jnp.maximum(sm, sm * NEG)
                w = jnp.exp(a - m)
                plsc.store_scatter(wx, [lane, jnp.full((16,), D + h, jnp.int32)], w)
                wtmp[pl.ds(16 + h * 16, 16)] = w

            @plsc.parallel_loop(0, 16, unroll=4)
            def _(l):
                e = g * 16 + l
                for h in range(H):
                    wb = plsc.load_gather(
                        wtmp, [jnp.full((16,), 16 + h * 16, jnp.int32) + l])
                    for p in range(2):
                        col = h * OC + p * 16
                        wx[e, pl.ds(col, 16)] = xjx[e, pl.ds(col, 16)] * wb

        pltpu.async_copy(wx, acc.at[ivbig.at[q]], sems, add=True)

        @pl.when(q == cnt - 1)
        def _():
            scatter_drain()

    @pl.loop(0, NSUP)
    def _(sup):
        cnt = jnp.where(sup < NSUP - 1, SUP, NCHUNK - (NSUP - 1) * SUP)
        roff = base + sup * SUP
        pltpu.sync_copy(ei_hbm.at[pl.ds(roff, SUP)], ivbig)
        pltpu.sync_copy(ej_hbm.at[pl.ds(roff, SUP)], jvbig)
        fire(0, 0)
        fire(1, 1)

        @pl.loop(0, (cnt + 1) // 2)
        def _(i):
            c0 = 2 * i
            wait_gathers(0)
            compute_and_scatter(0, c0, cnt)

            @pl.when(c0 + 2 < cnt)
            def _():
                fire(c0 + 2, 0)

            @pl.when(c0 + 1 < cnt)
            def _():
                wait_gathers(1)
                compute_and_scatter(1, c0 + 1, cnt)

                @pl.when(c0 + 3 < cnt)
                def _():
                    fire(c0 + 3, 1)

    plsc.subcore_barrier()

    r0 = sid * RPT

    for g in range(5):
        pltpu.async_copy(acc.at[pl.ds(r0 + g * RZB, RZB)],
                         out_hbm.at[cid, pl.ds(r0 + g * RZB, RZB)], sems)

    for g in range(5):
        pltpu.make_async_copy(acc.at[pl.ds(r0 + g * RZB, RZB)],
                              out_hbm.at[cid, pl.ds(r0 + g * RZB, RZB)],
                              sems).wait()


def _edge_kernel(x, s_tab, ei, ej, tmax16):
    mesh = plsc.VectorSubcoreMesh(core_axis_name="c", subcore_axis_name="s")
    run = functools.partial(
        pl.kernel,
        out_type=jax.ShapeDtypeStruct((NC, N, ROW), jnp.float32),
        mesh=mesh,
        compiler_params=pltpu.CompilerParams(
            use_tc_tiling_on_sc=False, needs_layout_passes=False),
        scratch_types=[
            pltpu.VMEM((SUP, K), jnp.int32),
            pltpu.VMEM((SUP, K), jnp.int32),
            pltpu.VMEM((K, 16), jnp.float32),
            pltpu.VMEM((K, 16), jnp.float32),
            pltpu.VMEM((K, D), jnp.float32),
            pltpu.VMEM((K, 16), jnp.float32),
            pltpu.VMEM((K, 16), jnp.float32),
            pltpu.VMEM((K, D), jnp.float32),
            pltpu.VMEM((K, ROW), jnp.float32),
            pltpu.VMEM((96,), jnp.float32),
            pltpu.VMEM((4, 16), jnp.float32),
            pltpu.VMEM_SHARED((N, ROW), jnp.float32),
            pltpu.SemaphoreType.DMA,
            pltpu.SemaphoreType.DMA,
            pltpu.SemaphoreType.DMA,
            pltpu.SemaphoreType.DMA,
            pltpu.SemaphoreType.DMA,
            pltpu.SemaphoreType.DMA,
            pltpu.SemaphoreType.DMA,
        ],
    )(_edge_body)
    return run(x, s_tab, ei, ej, tmax16)


def _norm_body(p_ref, o_ref):
    p = p_ref[...]
    q = p[0] + p[1]
    parts = []
    for h in range(H):
        den = q[:, D + h][:, None] + 1e-16
        parts.append(q[:, h * OC:(h + 1) * OC] / den)
    o_ref[...] = jnp.concatenate(parts, axis=1)


def _norm(p):
    R = 1000
    return pl.pallas_call(
        _norm_body,
        grid=(N // R,),
        in_specs=[pl.BlockSpec((2, R, ROW), lambda b: (0, b, 0))],
        out_specs=pl.BlockSpec((R, D), lambda b: (b, 0)),
        out_shape=jax.ShapeDtypeStruct((N, D), jnp.float32),
    )(p)


def kernel(x_tangent0, edges, W):
    x = x_tangent0
    w1 = W[0, :OC]
    w2 = W[0, OC:]
    eye = jnp.eye(H, dtype=jnp.float32)
    wf_s = jnp.kron(eye, w1[:, None])
    wf_t = jnp.kron(eye, w2[:, None])
    wfull = jnp.concatenate(
        [jnp.zeros((D, 8), jnp.float32), wf_s, wf_t], axis=1)

    s_tab = _scores(x, wfull)
    tmax = jnp.max(s_tab[:, 3 * H:4 * H], axis=0)
    tmax16 = jnp.broadcast_to(tmax[:, None], (H, 16))

    pad = jnp.zeros((SUP - (NCHUNK % SUP or SUP), K), jnp.int32)
    ei = jnp.concatenate([edges[0].astype(jnp.int32).reshape(E // K, K), pad])
    ej = jnp.concatenate([edges[1].astype(jnp.int32).reshape(E // K, K), pad])
    p = _edge_kernel(x, s_tab, ei, ej, tmax16)
    return _norm(p)
